# baseline (speedup 1.0000x reference)
import jax
import jax.numpy as jnp
from jax.experimental import pallas as pl
from jax.experimental.pallas import tpu as pltpu

_ROWS = 2048
_K = 3
_INT_MIN = -2147483648


def _router_kernel(x_ref, w_ref, b_ref, mask_ref, prob_ref, imp_ref):
    logits = jnp.dot(x_ref[...], w_ref[...],
                     preferred_element_type=jnp.float32,
                     precision=jax.lax.Precision.DEFAULT) + b_ref[...]

    m = jnp.max(logits, axis=-1, keepdims=True)
    e = jnp.exp(logits - m)
    prob = e / jnp.sum(e, axis=-1, keepdims=True)
    prob_ref[...] = prob

    @pl.when(pl.program_id(0) == 0)
    def _init():
        imp_ref[...] = jnp.zeros_like(imp_ref)

    imp_ref[...] += jnp.sum(prob, axis=0, keepdims=True)

    n_e = logits.shape[-1]
    cols = jax.lax.broadcasted_iota(jnp.int32, logits.shape, 1)
    bits = logits.view(jnp.int32)
    key = bits ^ jax.lax.shift_right_arithmetic(
        bits, 31).__and__(jnp.int32(0x7FFFFFFF))
    hit_any = None
    for _ in range(_K):
        mx = jnp.max(key, axis=-1, keepdims=True)
        cand = jnp.where(key == mx, cols, n_e)
        sel = jnp.min(cand, axis=-1, keepdims=True)
        hit = cols == sel
        hit_any = hit if hit_any is None else (hit_any | hit)
        key = jnp.where(hit, _INT_MIN, key)
    mask_ref[...] = hit_any.astype(jnp.float32)


def kernel(x, W, b):
    x = x.reshape(x.shape[0], -1)
    n, d = x.shape
    n_e = W.shape[1]
    grid = n // _ROWS
    mask, prob, imp = pl.pallas_call(
        _router_kernel,
        grid=(grid,),
        in_specs=[
            pl.BlockSpec((_ROWS, d), lambda i: (i, 0)),
            pl.BlockSpec((d, n_e), lambda i: (0, 0)),
            pl.BlockSpec((1, n_e), lambda i: (0, 0)),
        ],
        out_specs=[
            pl.BlockSpec((_ROWS, n_e), lambda i: (i, 0)),
            pl.BlockSpec((_ROWS, n_e), lambda i: (i, 0)),
            pl.BlockSpec((1, n_e), lambda i: (0, 0)),
        ],
        out_shape=[
            jax.ShapeDtypeStruct((n, n_e), jnp.float32),
            jax.ShapeDtypeStruct((n, n_e), jnp.float32),
            jax.ShapeDtypeStruct((1, n_e), jnp.float32),
        ],
        compiler_params=pltpu.CompilerParams(
            dimension_semantics=("arbitrary",)),
    )(x, W, b.reshape(1, -1))
    imp = imp.reshape(-1)
    return mask, prob, imp, imp


# speedup vs baseline: 1.1276x; 1.1276x over previous
import jax
import jax.numpy as jnp
from jax.experimental import pallas as pl
from jax.experimental.pallas import tpu as pltpu

_ROWS = 2048
_K = 3
_INT_MIN = -2147483648


def _router_kernel(x_ref, w_ref, b_ref, mask_ref, prob_ref, imp_ref):
    logits = jnp.dot(x_ref[...], w_ref[...],
                     preferred_element_type=jnp.float32,
                     precision=jax.lax.Precision.DEFAULT) + b_ref[...]

    m = jnp.max(logits, axis=-1, keepdims=True)
    e = jnp.exp(logits - m)
    prob = e / jnp.sum(e, axis=-1, keepdims=True)
    prob_ref[...] = prob

    @pl.when(pl.program_id(0) == 0)
    def _init():
        imp_ref[...] = jnp.zeros_like(imp_ref)

    imp_ref[...] += jnp.sum(prob, axis=0, keepdims=True)

    n_e = logits.shape[-1]
    cols = jax.lax.broadcasted_iota(
        jnp.int32, logits.shape, 1).astype(jnp.float32)
    big = jnp.float32(n_e)
    work = logits
    hit_any = None
    for _ in range(_K):
        mx = jnp.max(work, axis=-1, keepdims=True)
        cand = jnp.where(work == mx, cols, big)
        sel = jnp.min(cand, axis=-1, keepdims=True)
        hit = cols == sel
        hit_any = hit if hit_any is None else (hit_any | hit)
        work = jnp.where(hit, -jnp.inf, work)
    mask_ref[...] = hit_any.astype(jnp.float32)


def kernel(x, W, b):
    x = x.reshape(x.shape[0], -1)
    n, d = x.shape
    n_e = W.shape[1]
    grid = n // _ROWS
    mask, prob, imp = pl.pallas_call(
        _router_kernel,
        grid=(grid,),
        in_specs=[
            pl.BlockSpec((_ROWS, d), lambda i: (i, 0)),
            pl.BlockSpec((d, n_e), lambda i: (0, 0)),
            pl.BlockSpec((1, n_e), lambda i: (0, 0)),
        ],
        out_specs=[
            pl.BlockSpec((_ROWS, n_e), lambda i: (i, 0)),
            pl.BlockSpec((_ROWS, n_e), lambda i: (i, 0)),
            pl.BlockSpec((1, n_e), lambda i: (0, 0)),
        ],
        out_shape=[
            jax.ShapeDtypeStruct((n, n_e), jnp.float32),
            jax.ShapeDtypeStruct((n, n_e), jnp.float32),
            jax.ShapeDtypeStruct((1, n_e), jnp.float32),
        ],
        compiler_params=pltpu.CompilerParams(
            dimension_semantics=("arbitrary",)),
    )(x, W, b.reshape(1, -1))
    imp = imp.reshape(-1)
    return mask, prob, imp, imp
